# baseline (speedup 1.0000x reference)
import functools

import jax
import jax.numpy as jnp
from jax import lax
from jax.experimental import pallas as pl
from jax.experimental.pallas import tpu as pltpu
from jax.experimental.pallas import tpu_sc as plsc

NUM_TASKS = 1000000
EMB_DIM = 64
SEQ_FEAT = 320
H0 = 256
H1 = 128
OUT_W = 85
B = 16384

_LANES = 16
_TILE_C = 128
_FULL_TILES = NUM_TASKS // _TILE_C
_TAIL_BASE = _FULL_TILES * _TILE_C
_CTILES = 3
_CHUNK_C = _CTILES * _TILE_C
_TPW = 245


@functools.lru_cache(maxsize=None)
def _gather_fn():
    info = plsc.get_sparse_core_info()
    nw = info.num_cores * info.num_subcores
    n_chunks = -(-_TPW // _CTILES)
    mesh = plsc.VectorSubcoreMesh(core_axis_name="c", subcore_axis_name="s")

    @functools.partial(
        pl.kernel,
        mesh=mesh,
        out_type=jax.ShapeDtypeStruct(((B + _LANES) * EMB_DIM,), jnp.float32),
        scratch_types=[
            pltpu.VMEM((B,), jnp.int32),
            pltpu.VMEM((B + _LANES,), jnp.int32),
            pltpu.VMEM((B + _LANES,), jnp.int32),
            pltpu.VMEM((B + _LANES,), jnp.int32),
            pltpu.VMEM((B + _LANES,), jnp.int32),
            pltpu.VMEM((EMB_DIM, _CHUNK_C), jnp.float32),
            pltpu.VMEM((_LANES, EMB_DIM), jnp.float32),
            pltpu.SemaphoreType.DMA,
            pltpu.SemaphoreType.DMA,
        ],
        compiler_params=pltpu.CompilerParams(needs_layout_passes=False),
    )
    def gather_k(tt_hbm, idx_hbm, out_hbm, idx_v, hit_v, hpos_v, tmp_v,
                 tpos_v, blk_v, stg_v, sem, osem):
        wid = lax.axis_index("s") * info.num_cores + lax.axis_index("c")
        pltpu.sync_copy(idx_hbm, idx_v)

        t0 = jnp.minimum(wid * _TPW, _FULL_TILES - _TPW)
        my_lo = t0 * _TILE_C
        my_hi = (t0 + _TPW) * _TILE_C
        lanes = lax.iota(jnp.int32, _LANES)

        def scan_all(g, cnt):
            vec = idx_v[pl.ds(g * _LANES, _LANES)]
            m = (vec >= my_lo) & (vec < my_hi)
            pref = plsc.cumsum(m.astype(jnp.int32))
            dest = jnp.where(m, cnt + pref - 1, B + lanes)
            plsc.store_scatter(hit_v, [dest], vec)
            pos = jnp.full((_LANES,), g * _LANES, jnp.int32) + lanes
            plsc.store_scatter(hpos_v, [dest], pos)
            return cnt + pref[_LANES - 1]

        cnt = lax.fori_loop(0, B // _LANES, scan_all, jnp.int32(0))
        ngroups = (cnt + _LANES - 1) // _LANES

        def chunk_body(ci, carry):
            c0 = (t0 + jnp.minimum(ci * _CTILES, _TPW - _CTILES)) * _TILE_C
            c0 = pl.multiple_of(c0, _TILE_C)
            pltpu.async_copy(
                tt_hbm.at[:, pl.ds(c0, _CHUNK_C)], blk_v, sem
            ).wait()

            def scan_hits(g, mcnt):
                vec = hit_v[pl.ds(g * _LANES, _LANES)]
                pos = hpos_v[pl.ds(g * _LANES, _LANES)]
                valid = (g * _LANES + lanes) < cnt
                m = (vec >= c0) & (vec < c0 + _CHUNK_C) & valid
                pref = plsc.cumsum(m.astype(jnp.int32))
                dest = jnp.where(m, mcnt + pref - 1, B + lanes)
                plsc.store_scatter(tmp_v, [dest], vec)
                plsc.store_scatter(tpos_v, [dest], pos)
                return mcnt + pref[_LANES - 1]

            mcnt = lax.fori_loop(0, ngroups, scan_hits, jnp.int32(0))

            def group_body(g, carry2):
                rem = mcnt - g * _LANES
                gm = lanes < rem
                colv = tmp_v[pl.ds(g * _LANES, _LANES)] - c0
                colv = jnp.where(gm, colv, 0)
                posv = tpos_v[pl.ds(g * _LANES, _LANES)]
                for j in range(EMB_DIM):
                    row = jnp.full((_LANES,), j, jnp.int32)
                    vals = plsc.load_gather(blk_v, [row, colv])
                    plsc.store_scatter(stg_v, [lanes, row], vals)
                ocopies = []
                for j in range(_LANES):
                    p = jnp.where(j < rem, posv[j], B)
                    ocopies.append(pltpu.async_copy(
                        stg_v.at[j],
                        out_hbm.at[pl.ds(p * EMB_DIM, EMB_DIM)],
                        osem,
                    ))
                for c in ocopies:
                    c.wait()
                return carry2

            ngx = (mcnt + _LANES - 1) // _LANES
            lax.fori_loop(0, ngx, group_body, jnp.int32(0))
            return carry

        lax.fori_loop(0, n_chunks, chunk_body, jnp.int32(0))

    return gather_k


def _mlp_body(seq_ref, emb_ref, tidx_ref, mrows_ref, w0s_ref, w0e_ref,
              s0_ref, t0_ref, w1_ref, s1_ref, t1_ref, wh_ref, bh_ref,
              out_ref):
    h = jnp.dot(seq_ref[...], w0s_ref[...], preferred_element_type=jnp.float32)
    tidx = tidx_ref[...]
    tail = tidx >= _TAIL_BASE
    onehot = jnp.where(
        (tidx - _TAIL_BASE) == lax.broadcasted_iota(jnp.int32, (seq_ref.shape[0], EMB_DIM), 1),
        1.0, 0.0)
    emb_tail = jnp.dot(onehot, mrows_ref[...], preferred_element_type=jnp.float32)
    emb = jnp.where(tail, emb_tail, emb_ref[...])
    h = h + jnp.dot(emb, w0e_ref[...], preferred_element_type=jnp.float32)
    h = h * s0_ref[...] + t0_ref[...]
    h = jnp.maximum(h, 0.0)
    h = jnp.dot(h, w1_ref[...], preferred_element_type=jnp.float32)
    h = h * s1_ref[...] + t1_ref[...]
    h = jnp.maximum(h, 0.0)
    o = jnp.dot(h, wh_ref[...], preferred_element_type=jnp.float32) + bh_ref[...]
    col = lax.broadcasted_iota(jnp.int32, o.shape, 1)
    elu1 = jnp.where(o > 0, o, jnp.exp(jnp.minimum(o, 0.0)) - 1.0) + (1.0 + 1e-7)
    out_ref[...] = jnp.where((col >= 40) & (col < 80), elu1, o)


def kernel(sequence_input, task_input, table, W0, b0, gamma0, beta0, mm0, mv0,
           W1, b1, gamma1, beta1, mm1, mv1, Wmu, bmu, Wsig, bsig, Wpi, bpi):
    seq_flat = jnp.reshape(sequence_input, (B, SEQ_FEAT))
    tt = jnp.transpose(table)
    mrows = table[_TAIL_BASE:]

    emb1d = _gather_fn()(tt, task_input)
    emb = jnp.reshape(emb1d[:B * EMB_DIM], (B, EMB_DIM))

    s0 = gamma0 / jnp.sqrt(mv0 + 1e-3)
    t0 = (b0 - mm0) * s0 + beta0
    s1 = gamma1 / jnp.sqrt(mv1 + 1e-3)
    t1 = (b1 - mm1) * s1 + beta1
    wh = jnp.concatenate([Wmu, Wsig, Wpi], axis=1)
    bh = jnp.concatenate([bmu, bsig, bpi], axis=0)

    tile = 512
    grid = (B // tile,)
    out = pl.pallas_call(
        _mlp_body,
        grid=grid,
        in_specs=[
            pl.BlockSpec((tile, SEQ_FEAT), lambda i: (i, 0)),
            pl.BlockSpec((tile, EMB_DIM), lambda i: (i, 0)),
            pl.BlockSpec((tile, 1), lambda i: (i, 0)),
            pl.BlockSpec((EMB_DIM, EMB_DIM), lambda i: (0, 0)),
            pl.BlockSpec((SEQ_FEAT, H0), lambda i: (0, 0)),
            pl.BlockSpec((EMB_DIM, H0), lambda i: (0, 0)),
            pl.BlockSpec((1, H0), lambda i: (0, 0)),
            pl.BlockSpec((1, H0), lambda i: (0, 0)),
            pl.BlockSpec((H0, H1), lambda i: (0, 0)),
            pl.BlockSpec((1, H1), lambda i: (0, 0)),
            pl.BlockSpec((1, H1), lambda i: (0, 0)),
            pl.BlockSpec((H1, OUT_W), lambda i: (0, 0)),
            pl.BlockSpec((1, OUT_W), lambda i: (0, 0)),
        ],
        out_specs=pl.BlockSpec((tile, OUT_W), lambda i: (i, 0)),
        out_shape=jax.ShapeDtypeStruct((B, OUT_W), jnp.float32),
    )(
        seq_flat, emb, task_input[:, None], mrows,
        W0[:SEQ_FEAT], W0[SEQ_FEAT:],
        s0[None, :], t0[None, :],
        W1, s1[None, :], t1[None, :],
        wh, bh[None, :],
    )
    return out


# speedup vs baseline: 2.5465x; 2.5465x over previous
import functools

import jax
import jax.numpy as jnp
import numpy as np
from jax import lax
from jax.experimental import pallas as pl
from jax.experimental.pallas import tpu as pltpu
from jax.experimental.pallas import tpu_sc as plsc

NUM_TASKS = 1000000
EMB_DIM = 64
SEQ_FEAT = 320
H0 = 256
H1 = 128
OUT_W = 85
B = 16384

_LANES = 16
_NW = 32
_TILE_C = 128
_FULL_TILES = NUM_TASKS // _TILE_C
_TAIL_BASE = _FULL_TILES * _TILE_C
_CTILES = 4
_CHUNK_C = _CTILES * _TILE_C
_TPW = 245
_NCHUNKS = -(-_TPW // _CTILES)

_T0 = np.minimum(np.arange(_NW) * _TPW, _FULL_TILES - _TPW)
_C0 = (_T0[:, None] + np.minimum(np.arange(_NCHUNKS) * _CTILES,
                                 _TPW - _CTILES)[None, :]) * _TILE_C
_PTR_PAD = 16
_NPTR = _NW + _NW * _NCHUNKS + _PTR_PAD


@functools.lru_cache(maxsize=None)
def _gather_fn():
    info = plsc.get_sparse_core_info()
    mesh = plsc.VectorSubcoreMesh(core_axis_name="c", subcore_axis_name="s")

    @functools.partial(
        pl.kernel,
        mesh=mesh,
        out_type=jax.ShapeDtypeStruct(((B + _NW * _LANES) * EMB_DIM,),
                                      jnp.float32),
        scratch_types=[
            pltpu.VMEM((B,), jnp.int32),
            pltpu.VMEM((B,), jnp.int32),
            pltpu.VMEM((_NPTR,), jnp.int32),
            pltpu.VMEM((2, EMB_DIM, _CHUNK_C), jnp.float32),
            pltpu.VMEM((_LANES, EMB_DIM), jnp.float32),
            pltpu.SemaphoreType.DMA,
            pltpu.SemaphoreType.DMA,
        ],
        compiler_params=pltpu.CompilerParams(needs_layout_passes=False),
    )
    def gather_k(tt_hbm, sidx_hbm, spos_hbm, ptrs_hbm, out_hbm,
                 sidx_v, spos_v, ptrs_v, blk_v, stg_v, sem, osem):
        wid = lax.axis_index("s") * info.num_cores + lax.axis_index("c")
        pltpu.sync_copy(sidx_hbm, sidx_v)
        pltpu.sync_copy(spos_hbm, spos_v)
        pltpu.sync_copy(ptrs_hbm, ptrs_v)

        t0 = jnp.minimum(wid * _TPW, _FULL_TILES - _TPW)
        lanes = lax.iota(jnp.int32, _LANES)
        ptr0 = ptrs_v[pl.ds(wid, _LANES)][0]

        def window(ci):
            c0 = (t0 + jnp.minimum(ci * _CTILES, _TPW - _CTILES)) * _TILE_C
            c0 = pl.multiple_of(c0, _TILE_C)
            return c0

        pltpu.async_copy(tt_hbm.at[:, pl.ds(window(0), _CHUNK_C)],
                         blk_v.at[0], sem)

        def chunk_body(ci, ptr):
            b = lax.rem(ci, 2)
            c0 = window(ci)
            pltpu.make_async_copy(
                tt_hbm.at[:, pl.ds(c0, _CHUNK_C)], blk_v.at[b], sem
            ).wait()
            cn = window(jnp.minimum(ci + 1, _NCHUNKS - 1))
            pltpu.async_copy(
                tt_hbm.at[:, pl.ds(cn, _CHUNK_C)],
                blk_v.at[lax.rem(ci + 1, 2)], sem)

            pe = ptrs_v[pl.ds(_NW + wid * _NCHUNKS + ci, _LANES)][0]

            def group_body(g, carry2):
                base = ptr + g * _LANES
                rem = pe - base
                gm = lanes < rem
                vec = sidx_v[pl.ds(base, _LANES)]
                posv = spos_v[pl.ds(base, _LANES)]
                colv = jnp.where(gm, vec - c0, 0)
                bvec = jnp.full((_LANES,), b, jnp.int32)
                for j in range(EMB_DIM):
                    row = jnp.full((_LANES,), j, jnp.int32)
                    vals = plsc.load_gather(blk_v, [bvec, row, colv])
                    plsc.store_scatter(stg_v, [lanes, row], vals)
                ocopies = []
                for j in range(_LANES):
                    dummy = B + wid * _LANES + j
                    p = jnp.where(j < rem, posv[j], dummy)
                    ocopies.append(pltpu.async_copy(
                        stg_v.at[j],
                        out_hbm.at[pl.ds(p * EMB_DIM, EMB_DIM)],
                        osem,
                    ))
                for c in ocopies:
                    c.wait()
                return carry2

            ngx = (pe - ptr + _LANES - 1) // _LANES
            lax.fori_loop(0, ngx, group_body, jnp.int32(0))
            return pe

        lax.fori_loop(0, _NCHUNKS, chunk_body, ptr0)
        pltpu.make_async_copy(
            tt_hbm.at[:, pl.ds(window(_NCHUNKS - 1), _CHUNK_C)],
            blk_v.at[lax.rem(jnp.int32(_NCHUNKS), 2)], sem
        ).wait()

    return gather_k


def _mlp_body(seq_ref, emb_ref, tidx_ref, mrows_ref, w0s_ref, w0e_ref,
              s0_ref, t0_ref, w1_ref, s1_ref, t1_ref, wh_ref, bh_ref,
              out_ref):
    h = jnp.dot(seq_ref[...], w0s_ref[...], preferred_element_type=jnp.float32)
    tidx = tidx_ref[...]
    tail = tidx >= _TAIL_BASE
    onehot = jnp.where(
        (tidx - _TAIL_BASE) == lax.broadcasted_iota(
            jnp.int32, (seq_ref.shape[0], EMB_DIM), 1),
        1.0, 0.0)
    emb_tail = jnp.dot(onehot, mrows_ref[...], preferred_element_type=jnp.float32)
    emb = jnp.where(tail, emb_tail, emb_ref[...])
    h = h + jnp.dot(emb, w0e_ref[...], preferred_element_type=jnp.float32)
    h = h * s0_ref[...] + t0_ref[...]
    h = jnp.maximum(h, 0.0)
    h = jnp.dot(h, w1_ref[...], preferred_element_type=jnp.float32)
    h = h * s1_ref[...] + t1_ref[...]
    h = jnp.maximum(h, 0.0)
    o = jnp.dot(h, wh_ref[...], preferred_element_type=jnp.float32) + bh_ref[...]
    col = lax.broadcasted_iota(jnp.int32, o.shape, 1)
    elu1 = jnp.where(o > 0, o, jnp.exp(jnp.minimum(o, 0.0)) - 1.0) + (1.0 + 1e-7)
    out_ref[...] = jnp.where((col >= 40) & (col < 80), elu1, o)


def kernel(sequence_input, task_input, table, W0, b0, gamma0, beta0, mm0, mv0,
           W1, b1, gamma1, beta1, mm1, mv1, Wmu, bmu, Wsig, bsig, Wpi, bpi):
    seq_flat = jnp.reshape(sequence_input, (B, SEQ_FEAT))
    tt = jnp.transpose(table)
    mrows = table[_TAIL_BASE:]

    sidx, spos = lax.sort((task_input, lax.iota(jnp.int32, B)), num_keys=1)
    bounds = jnp.concatenate([
        jnp.asarray(_T0 * _TILE_C, jnp.int32),
        jnp.asarray(_C0.reshape(-1) + _CHUNK_C, jnp.int32),
    ])
    ptrs = jnp.searchsorted(sidx, bounds, side="left").astype(jnp.int32)
    ptrs = jnp.concatenate([ptrs, jnp.zeros((_PTR_PAD,), jnp.int32)])

    emb1d = _gather_fn()(tt, sidx, spos, ptrs)
    emb = jnp.reshape(emb1d[:B * EMB_DIM], (B, EMB_DIM))

    s0 = gamma0 / jnp.sqrt(mv0 + 1e-3)
    t0 = (b0 - mm0) * s0 + beta0
    s1 = gamma1 / jnp.sqrt(mv1 + 1e-3)
    t1 = (b1 - mm1) * s1 + beta1
    wh = jnp.concatenate([Wmu, Wsig, Wpi], axis=1)
    bh = jnp.concatenate([bmu, bsig, bpi], axis=0)

    tile = 512
    grid = (B // tile,)
    out = pl.pallas_call(
        _mlp_body,
        grid=grid,
        in_specs=[
            pl.BlockSpec((tile, SEQ_FEAT), lambda i: (i, 0)),
            pl.BlockSpec((tile, EMB_DIM), lambda i: (i, 0)),
            pl.BlockSpec((tile, 1), lambda i: (i, 0)),
            pl.BlockSpec((EMB_DIM, EMB_DIM), lambda i: (0, 0)),
            pl.BlockSpec((SEQ_FEAT, H0), lambda i: (0, 0)),
            pl.BlockSpec((EMB_DIM, H0), lambda i: (0, 0)),
            pl.BlockSpec((1, H0), lambda i: (0, 0)),
            pl.BlockSpec((1, H0), lambda i: (0, 0)),
            pl.BlockSpec((H0, H1), lambda i: (0, 0)),
            pl.BlockSpec((1, H1), lambda i: (0, 0)),
            pl.BlockSpec((1, H1), lambda i: (0, 0)),
            pl.BlockSpec((H1, OUT_W), lambda i: (0, 0)),
            pl.BlockSpec((1, OUT_W), lambda i: (0, 0)),
        ],
        out_specs=pl.BlockSpec((tile, OUT_W), lambda i: (i, 0)),
        out_shape=jax.ShapeDtypeStruct((B, OUT_W), jnp.float32),
    )(
        seq_flat, emb, task_input[:, None], mrows,
        W0[:SEQ_FEAT], W0[SEQ_FEAT:],
        s0[None, :], t0[None, :],
        W1, s1[None, :], t1[None, :],
        wh, bh[None, :],
    )
    return out


# speedup vs baseline: 2.8633x; 1.1244x over previous
import functools

import jax
import jax.numpy as jnp
import numpy as np
from jax import lax
from jax.experimental import pallas as pl
from jax.experimental.pallas import tpu as pltpu
from jax.experimental.pallas import tpu_sc as plsc

NUM_TASKS = 1000000
EMB_DIM = 64
SEQ_FEAT = 320
H0 = 256
H1 = 128
OUT_W = 85
B = 16384

_LANES = 16
_NW = 32
_TILE_C = 128
_FULL_TILES = NUM_TASKS // _TILE_C
_TAIL_BASE = _FULL_TILES * _TILE_C
_CTILES = 5
_CHUNK_C = _CTILES * _TILE_C
_TPW = 245
_NCHUNKS = -(-_TPW // _CTILES)

_T0 = np.minimum(np.arange(_NW) * _TPW, _FULL_TILES - _TPW)
_C0 = (_T0[:, None] + np.minimum(np.arange(_NCHUNKS) * _CTILES,
                                 _TPW - _CTILES)[None, :]) * _TILE_C
_PTR_PAD = 16
_NPTR = _NW + _NW * _NCHUNKS + _PTR_PAD


@functools.lru_cache(maxsize=None)
def _gather_fn():
    info = plsc.get_sparse_core_info()
    mesh = plsc.VectorSubcoreMesh(core_axis_name="c", subcore_axis_name="s")

    @functools.partial(
        pl.kernel,
        mesh=mesh,
        out_type=jax.ShapeDtypeStruct(((B + _NW * _LANES) * EMB_DIM,),
                                      jnp.float32),
        scratch_types=[
            pltpu.VMEM((B,), jnp.int32),
            pltpu.VMEM((B,), jnp.int32),
            pltpu.VMEM((_NPTR,), jnp.int32),
            pltpu.VMEM((2, EMB_DIM, _CHUNK_C), jnp.float32),
            pltpu.VMEM((_LANES, EMB_DIM), jnp.float32),
            pltpu.SemaphoreType.DMA,
            pltpu.SemaphoreType.DMA,
        ],
        compiler_params=pltpu.CompilerParams(needs_layout_passes=False),
    )
    def gather_k(tt_hbm, sidx_hbm, spos_hbm, ptrs_hbm, out_hbm,
                 sidx_v, spos_v, ptrs_v, blk_v, stg_v, sem, osem):
        wid = lax.axis_index("s") * info.num_cores + lax.axis_index("c")
        pltpu.sync_copy(sidx_hbm, sidx_v)
        pltpu.sync_copy(spos_hbm, spos_v)
        pltpu.sync_copy(ptrs_hbm, ptrs_v)

        t0 = jnp.minimum(wid * _TPW, _FULL_TILES - _TPW)
        lanes = lax.iota(jnp.int32, _LANES)
        ptr0 = ptrs_v[pl.ds(wid, _LANES)][0]

        def window(ci):
            c0 = (t0 + jnp.minimum(ci * _CTILES, _TPW - _CTILES)) * _TILE_C
            c0 = pl.multiple_of(c0, _TILE_C)
            return c0

        pltpu.async_copy(tt_hbm.at[:, pl.ds(window(0), _CHUNK_C)],
                         blk_v.at[0], sem)

        def chunk_body(ci, ptr):
            b = lax.rem(ci, 2)
            c0 = window(ci)
            pltpu.make_async_copy(
                tt_hbm.at[:, pl.ds(c0, _CHUNK_C)], blk_v.at[b], sem
            ).wait()
            cn = window(jnp.minimum(ci + 1, _NCHUNKS - 1))
            pltpu.async_copy(
                tt_hbm.at[:, pl.ds(cn, _CHUNK_C)],
                blk_v.at[lax.rem(ci + 1, 2)], sem)

            pe = ptrs_v[pl.ds(_NW + wid * _NCHUNKS + ci, _LANES)][0]

            def group_body(g, carry2):
                base = ptr + g * _LANES
                rem = pe - base
                gm = lanes < rem
                vec = sidx_v[pl.ds(base, _LANES)]
                posv = spos_v[pl.ds(base, _LANES)]
                colv = jnp.where(gm, vec - c0, 0)
                bvec = jnp.full((_LANES,), b, jnp.int32)
                for j in range(EMB_DIM):
                    row = jnp.full((_LANES,), j, jnp.int32)
                    vals = plsc.load_gather(blk_v, [bvec, row, colv])
                    plsc.store_scatter(stg_v, [lanes, row], vals)
                ocopies = []
                for j in range(_LANES):
                    dummy = B + wid * _LANES + j
                    p = jnp.where(j < rem, posv[j], dummy)
                    ocopies.append(pltpu.async_copy(
                        stg_v.at[j],
                        out_hbm.at[pl.ds(p * EMB_DIM, EMB_DIM)],
                        osem,
                    ))
                for c in ocopies:
                    c.wait()
                return carry2

            ngx = (pe - ptr + _LANES - 1) // _LANES
            lax.fori_loop(0, ngx, group_body, jnp.int32(0))
            return pe

        lax.fori_loop(0, _NCHUNKS, chunk_body, ptr0)
        pltpu.make_async_copy(
            tt_hbm.at[:, pl.ds(window(_NCHUNKS - 1), _CHUNK_C)],
            blk_v.at[lax.rem(jnp.int32(_NCHUNKS), 2)], sem
        ).wait()

    return gather_k


def _mlp_body(seq_ref, emb_ref, tidx_ref, mrows_ref, w0s_ref, w0e_ref,
              s0_ref, t0_ref, w1_ref, s1_ref, t1_ref, wh_ref, bh_ref,
              out_ref):
    h = lax.dot_general(seq_ref[...], w0s_ref[...],
                        (((0,), (0,)), ((), ())),
                        preferred_element_type=jnp.float32)
    tidx = tidx_ref[...]
    tail = tidx >= _TAIL_BASE
    onehot = jnp.where(
        (tidx - _TAIL_BASE) == lax.broadcasted_iota(
            jnp.int32, (emb_ref.shape[0], EMB_DIM), 1),
        1.0, 0.0)
    emb_tail = jnp.dot(onehot, mrows_ref[...], preferred_element_type=jnp.float32)
    emb = jnp.where(tail, emb_tail, emb_ref[...])
    h = h + jnp.dot(emb, w0e_ref[...], preferred_element_type=jnp.float32)
    h = h * s0_ref[...] + t0_ref[...]
    h = jnp.maximum(h, 0.0)
    h = jnp.dot(h, w1_ref[...], preferred_element_type=jnp.float32)
    h = h * s1_ref[...] + t1_ref[...]
    h = jnp.maximum(h, 0.0)
    o = jnp.dot(h, wh_ref[...], preferred_element_type=jnp.float32) + bh_ref[...]
    col = lax.broadcasted_iota(jnp.int32, o.shape, 1)
    elu1 = jnp.where(o > 0, o, jnp.exp(jnp.minimum(o, 0.0)) - 1.0) + (1.0 + 1e-7)
    out_ref[...] = jnp.where((col >= 40) & (col < 80), elu1, o)


def kernel(sequence_input, task_input, table, W0, b0, gamma0, beta0, mm0, mv0,
           W1, b1, gamma1, beta1, mm1, mv1, Wmu, bmu, Wsig, bsig, Wpi, bpi):
    seq_t = jnp.transpose(jnp.reshape(sequence_input, (B, SEQ_FEAT)))
    tt = jnp.transpose(table)
    mrows = table[_TAIL_BASE:]

    sidx, spos = lax.sort((task_input, lax.iota(jnp.int32, B)), num_keys=1)
    bounds = jnp.concatenate([
        jnp.asarray(_T0 * _TILE_C, jnp.int32),
        jnp.asarray(_C0.reshape(-1) + _CHUNK_C, jnp.int32),
    ])
    ptrs = jnp.searchsorted(sidx, bounds, side="left").astype(jnp.int32)
    ptrs = jnp.concatenate([ptrs, jnp.zeros((_PTR_PAD,), jnp.int32)])

    emb1d = _gather_fn()(tt, sidx, spos, ptrs)
    emb = jnp.reshape(emb1d[:B * EMB_DIM], (B, EMB_DIM))

    s0 = gamma0 / jnp.sqrt(mv0 + 1e-3)
    t0 = (b0 - mm0) * s0 + beta0
    s1 = gamma1 / jnp.sqrt(mv1 + 1e-3)
    t1 = (b1 - mm1) * s1 + beta1
    wh = jnp.concatenate([Wmu, Wsig, Wpi], axis=1)
    bh = jnp.concatenate([bmu, bsig, bpi], axis=0)

    tile = 1024
    grid = (B // tile,)
    out = pl.pallas_call(
        _mlp_body,
        grid=grid,
        in_specs=[
            pl.BlockSpec((SEQ_FEAT, tile), lambda i: (0, i)),
            pl.BlockSpec((tile, EMB_DIM), lambda i: (i, 0)),
            pl.BlockSpec((tile, 1), lambda i: (i, 0)),
            pl.BlockSpec((EMB_DIM, EMB_DIM), lambda i: (0, 0)),
            pl.BlockSpec((SEQ_FEAT, H0), lambda i: (0, 0)),
            pl.BlockSpec((EMB_DIM, H0), lambda i: (0, 0)),
            pl.BlockSpec((1, H0), lambda i: (0, 0)),
            pl.BlockSpec((1, H0), lambda i: (0, 0)),
            pl.BlockSpec((H0, H1), lambda i: (0, 0)),
            pl.BlockSpec((1, H1), lambda i: (0, 0)),
            pl.BlockSpec((1, H1), lambda i: (0, 0)),
            pl.BlockSpec((H1, OUT_W), lambda i: (0, 0)),
            pl.BlockSpec((1, OUT_W), lambda i: (0, 0)),
        ],
        out_specs=pl.BlockSpec((tile, OUT_W), lambda i: (i, 0)),
        out_shape=jax.ShapeDtypeStruct((B, OUT_W), jnp.float32),
    )(
        seq_t, emb, task_input[:, None], mrows,
        W0[:SEQ_FEAT], W0[SEQ_FEAT:],
        s0[None, :], t0[None, :],
        W1, s1[None, :], t1[None, :],
        wh, bh[None, :],
    )
    return out
